# baseline (speedup 1.0000x reference)
import functools

import jax
import jax.numpy as jnp
from jax import lax
from jax.experimental import pallas as pl
from jax.experimental.pallas import tpu as pltpu
from jax.experimental.pallas import tpu_sc as plsc

_TCA = 25
_GCA = 9
_KS = (24, 23, 22)
_ITERS = 10
_MAXK = 24
_VOCAB = 1000000
_DIM = 32
_NSCAN = _VOCAB - 1
_PACK = 128 // _DIM
_PROWS = _VOCAB // _PACK
_BLK = 2048
_NBLK = (_PROWS + _BLK - 1) // _BLK
_BIG = 3.0e38


def _kmeans_run(data, pmask, k):
    jj = lax.broadcasted_iota(jnp.int32, (1, _MAXK), 1)
    kmask = jj < k
    cents0 = data[:_MAXK]
    ones_col = jnp.ones((32, 1), jnp.float32)

    def labels_of(cents):
        diff = data[:, None, :] - cents[None, :, :]
        d2 = jnp.sum(diff * diff, axis=-1)
        d2 = jnp.where(kmask, d2, _BIG)
        m = jnp.min(d2, axis=1, keepdims=True)
        return jnp.min(jnp.where(d2 == m, jj, _MAXK), axis=1, keepdims=True)

    def body(_, cents):
        lab = labels_of(cents)
        onehot = ((lab == jj) & pmask).astype(jnp.float32)
        counts = lax.dot_general(onehot, ones_col,
                                 (((0,), (0,)), ((), ())))
        sums = lax.dot_general(onehot, data,
                               (((0,), (0,)), ((), ())))
        newc = sums / jnp.maximum(counts, 1.0)
        return jnp.where(counts > 0, newc, cents)

    cents = lax.fori_loop(0, _ITERS, body, cents0)
    lab = labels_of(cents)
    onehot = ((lab == jj) & pmask).astype(jnp.float32)
    ii = lax.broadcasted_iota(jnp.int32, (32, 1), 0)
    good = jnp.sum(onehot * (ii < _GCA), axis=0, keepdims=True)
    bad = jnp.sum(onehot * ((ii >= _GCA) & pmask), axis=0, keepdims=True)
    sizes = jnp.where((bad == 0.0) & (good > 0.0), good, 0.0)
    return sizes, cents


def _tc_kmeans_body(idxp_ref, grp_ref, emb_ref, cent_ref, len_ref,
                    rows_v, sem):
    copies = [
        pltpu.make_async_copy(emb_ref.at[pl.ds(idxp_ref[j], 1)],
                              rows_v.at[pl.ds(j, 1)], sem)
        for j in range(_TCA)
    ]
    for c in copies:
        c.start()
    for c in copies:
        c.wait()
    pmask = lax.broadcasted_iota(jnp.int32, (32, 1), 0) < _TCA
    grp = grp_ref[...]
    gath = rows_v[...]
    data = jnp.zeros((32, _DIM), jnp.float32)
    for g in range(_PACK):
        data = data + jnp.where((grp == g) & pmask,
                                gath[:, g * _DIM:(g + 1) * _DIM], 0.0)
    jj = lax.broadcasted_iota(jnp.int32, (1, _MAXK), 1)

    ms, cents_sel = [], []
    for k in _KS:
        sizes, cents = _kmeans_run(data, pmask, k)
        m = jnp.max(sizes)
        arg = jnp.min(jnp.where(sizes == m, jj, _MAXK))
        oh = (jj == arg).astype(jnp.float32)
        csel = lax.dot_general(oh, cents, (((1,), (0,)), ((), ())))
        ms.append(m)
        cents_sel.append(csel)

    gm = jnp.maximum(jnp.maximum(ms[0], ms[1]), ms[2])
    s0 = ms[0] == gm
    s1 = (ms[1] == gm) & (~s0)
    s2 = (ms[2] == gm) & (~s0) & (~s1)
    centroid = (jnp.where(s0, 1.0, 0.0) * cents_sel[0]
                + jnp.where(s1, 1.0, 0.0) * cents_sel[1]
                + jnp.where(s2, 1.0, 0.0) * cents_sel[2])
    r = lax.broadcasted_iota(jnp.int32, (_DIM, 128), 0)
    c = lax.broadcasted_iota(jnp.int32, (_DIM, 128), 1)
    tiler = (r == (c % _DIM)).astype(jnp.float32)
    cent_ref[...] = lax.dot_general(centroid, tiler, (((1,), (0,)), ((), ())))
    len_ref[0, 0] = gm.astype(jnp.int32)


_tc_kmeans = pl.pallas_call(
    _tc_kmeans_body,
    in_specs=[
        pl.BlockSpec(memory_space=pltpu.SMEM),
        pl.BlockSpec(memory_space=pltpu.VMEM),
        pl.BlockSpec(memory_space=pl.ANY),
    ],
    out_shape=(
        jax.ShapeDtypeStruct((1, 128), jnp.float32),
        jax.ShapeDtypeStruct((1, 1), jnp.int32),
    ),
    out_specs=(
        pl.BlockSpec(memory_space=pltpu.VMEM),
        pl.BlockSpec(memory_space=pltpu.SMEM),
    ),
    scratch_shapes=[
        pltpu.VMEM((32, 128), jnp.float32),
        pltpu.SemaphoreType.DMA,
    ],
)


def _block_d2(x, cent_t):
    z = x - cent_t
    z2 = z * z
    r = lax.broadcasted_iota(jnp.int32, (128, _PACK), 0)
    c = lax.broadcasted_iota(jnp.int32, (128, _PACK), 1)
    seg = (r // _DIM == c).astype(jnp.float32)
    return lax.dot_general(z2, seg, (((1,), (0,)), ((), ())))


def _tc_scan_body(cent_ref, emb_ref, min_ref, blk_ref, minv, mini):
    i = pl.program_id(0)

    @pl.when(i == 0)
    def _():
        minv[0] = jnp.float32(_BIG)
        mini[0] = 0

    d2 = _block_d2(emb_ref[...], cent_ref[...])

    @pl.when(i < _NBLK - 1)
    def _():
        bmin = jnp.min(d2)

        @pl.when(bmin < minv[0])
        def _():
            minv[0] = bmin
            mini[0] = i

    @pl.when(i == _NBLK - 1)
    def _():
        rr = lax.broadcasted_iota(jnp.int32, (_BLK, _PACK), 0)
        qq = lax.broadcasted_iota(jnp.int32, (_BLK, _PACK), 1)
        rows = (i * _BLK + rr) * _PACK + qq
        bmin = jnp.min(jnp.where(rows < _NSCAN, d2, _BIG))

        @pl.when(bmin < minv[0])
        def _():
            minv[0] = bmin
            mini[0] = i

        min_ref[0, 0] = minv[0]
        blk_ref[0, 0] = mini[0]


_tc_scan = pl.pallas_call(
    _tc_scan_body,
    grid=(_NBLK,),
    in_specs=[
        pl.BlockSpec((1, 128), lambda i: (0, 0)),
        pl.BlockSpec((_BLK, 128), lambda i: (i, 0)),
    ],
    out_specs=(
        pl.BlockSpec((1, 1), lambda i: (0, 0), memory_space=pltpu.SMEM),
        pl.BlockSpec((1, 1), lambda i: (0, 0), memory_space=pltpu.SMEM),
    ),
    out_shape=(
        jax.ShapeDtypeStruct((1, 1), jnp.float32),
        jax.ShapeDtypeStruct((1, 1), jnp.int32),
    ),
    scratch_shapes=[
        pltpu.SMEM((1,), jnp.float32),
        pltpu.SMEM((1,), jnp.int32),
    ],
)


def _tc_arg_body(cent_ref, min_ref, blk_ref, emb_ref, idx_ref, dist_ref,
                 xv, sem):
    start = jnp.minimum(blk_ref[0, 0] * _BLK, _PROWS - _BLK)
    pltpu.make_async_copy(emb_ref.at[pl.ds(start, _BLK)], xv, sem).start()
    pltpu.make_async_copy(emb_ref.at[pl.ds(start, _BLK)], xv, sem).wait()
    d2 = _block_d2(xv[...], cent_ref[...])
    rr = lax.broadcasted_iota(jnp.int32, (_BLK, _PACK), 0)
    qq = lax.broadcasted_iota(jnp.int32, (_BLK, _PACK), 1)
    rows = (start + rr) * _PACK + qq
    d2 = jnp.where(rows < _NSCAN, d2, _BIG)
    gmin = min_ref[0, 0]
    idx_ref[0, 0] = jnp.min(jnp.where(d2 == gmin, rows, _VOCAB))
    dist_ref[0, 0] = jnp.sqrt(gmin)


_tc_arg = pl.pallas_call(
    _tc_arg_body,
    in_specs=[
        pl.BlockSpec(memory_space=pltpu.VMEM),
        pl.BlockSpec(memory_space=pltpu.SMEM),
        pl.BlockSpec(memory_space=pltpu.SMEM),
        pl.BlockSpec(memory_space=pl.ANY),
    ],
    out_shape=(
        jax.ShapeDtypeStruct((1, 1), jnp.int32),
        jax.ShapeDtypeStruct((1, 1), jnp.float32),
    ),
    out_specs=(
        pl.BlockSpec(memory_space=pltpu.SMEM),
        pl.BlockSpec(memory_space=pltpu.SMEM),
    ),
    scratch_shapes=[
        pltpu.VMEM((_BLK, 128), jnp.float32),
        pltpu.SemaphoreType.DMA,
    ],
)


def kernel(embeddings, good_idx, bad_idx):
    idx = jnp.concatenate([
        good_idx.astype(jnp.int32),
        bad_idx.astype(jnp.int32),
        jnp.zeros((32 - _TCA,), jnp.int32),
    ])
    emb_p = embeddings.reshape(_PROWS, 128)
    idx_p = idx // _PACK
    grp = (idx % _PACK).reshape(32, 1)
    cent_t, clue_len = _tc_kmeans(idx_p, grp, emb_p)
    gmin, gblk = _tc_scan(cent_t, emb_p)
    clue_idx, min_dist = _tc_arg(cent_t, gmin, gblk, emb_p)
    return clue_idx[0, 0], clue_len[0, 0], min_dist[0, 0]


# speedup vs baseline: 1.3754x; 1.3754x over previous
import jax
import jax.numpy as jnp
from jax import lax
from jax.experimental import pallas as pl
from jax.experimental.pallas import tpu as pltpu

_TCA = 25
_GCA = 9
_KS = (24, 23, 22)
_ITERS = 10
_MAXK = 24
_VOCAB = 1000000
_DIM = 32
_NSCAN = _VOCAB - 1
_BLK = 16384
_NBLK = (_VOCAB + _BLK - 1) // _BLK
_BIG = 3.0e38


def _kmeans_run(data, pmask, k):
    jj = lax.broadcasted_iota(jnp.int32, (1, _MAXK), 1)
    kmask = jj < k
    cents0 = data[:_MAXK]
    ones_col = jnp.ones((32, 1), jnp.float32)

    def labels_of(cents):
        diff = data[:, None, :] - cents[None, :, :]
        d2 = jnp.sum(diff * diff, axis=-1)
        d2 = jnp.where(kmask, d2, _BIG)
        m = jnp.min(d2, axis=1, keepdims=True)
        return jnp.min(jnp.where(d2 == m, jj, _MAXK), axis=1, keepdims=True)

    def body(_, cents):
        lab = labels_of(cents)
        onehot = ((lab == jj) & pmask).astype(jnp.float32)
        counts = lax.dot_general(onehot, ones_col,
                                 (((0,), (0,)), ((), ())))
        sums = lax.dot_general(onehot, data,
                               (((0,), (0,)), ((), ())))
        newc = sums / jnp.maximum(counts, 1.0)
        return jnp.where(counts > 0, newc, cents)

    cents = lax.fori_loop(0, _ITERS, body, cents0)
    lab = labels_of(cents)
    onehot = ((lab == jj) & pmask).astype(jnp.float32)
    ii = lax.broadcasted_iota(jnp.int32, (32, 1), 0)
    good = jnp.sum(onehot * (ii < _GCA), axis=0, keepdims=True)
    bad = jnp.sum(onehot * ((ii >= _GCA) & pmask), axis=0, keepdims=True)
    sizes = jnp.where((bad == 0.0) & (good > 0.0), good, 0.0)
    return sizes, cents


def _kmeans_centroid(data, pmask):
    jj = lax.broadcasted_iota(jnp.int32, (1, _MAXK), 1)
    ms, cents_sel = [], []
    for k in _KS:
        sizes, cents = _kmeans_run(data, pmask, k)
        m = jnp.max(sizes)
        arg = jnp.min(jnp.where(sizes == m, jj, _MAXK))
        oh = (jj == arg).astype(jnp.float32)
        csel = lax.dot_general(oh, cents, (((1,), (0,)), ((), ())))
        ms.append(m)
        cents_sel.append(csel)

    gm = jnp.maximum(jnp.maximum(ms[0], ms[1]), ms[2])
    s0 = ms[0] == gm
    s1 = (ms[1] == gm) & (~s0)
    s2 = (ms[2] == gm) & (~s0) & (~s1)
    centroid = (jnp.where(s0, 1.0, 0.0) * cents_sel[0]
                + jnp.where(s1, 1.0, 0.0) * cents_sel[1]
                + jnp.where(s2, 1.0, 0.0) * cents_sel[2])
    return centroid, gm


def _block_d2t(x, cent):
    z = x - cent
    z2 = z * z
    ones = jnp.ones((1, _DIM), jnp.float32)
    return lax.dot_general(ones, z2, (((1,), (1,)), ((), ())))


def _body(idx_ref, emb_ref, len_ref, idxo_ref, dist_ref,
          bufs, rows_v, cent_v, minv, mini, sems, gsem):
    i = pl.program_id(0)

    def block_copy(blk, buf):
        st = jnp.minimum(blk * _BLK, _VOCAB - _BLK)
        return pltpu.make_async_copy(
            emb_ref.at[pl.ds(st, _BLK)], bufs.at[buf], sems.at[buf])

    @pl.when(i == 0)
    def _():
        block_copy(0, 0).start()
        gathers = [
            pltpu.make_async_copy(emb_ref.at[pl.ds(idx_ref[j], 1)],
                                  rows_v.at[pl.ds(j, 1)], gsem)
            for j in range(_TCA)
        ]
        for g in gathers:
            g.start()
        for g in gathers:
            g.wait()
        pmask = lax.broadcasted_iota(jnp.int32, (32, 1), 0) < _TCA
        data = jnp.where(pmask, rows_v[...], 0.0)
        centroid, gm = _kmeans_centroid(data, pmask)
        cent_v[...] = centroid
        len_ref[0, 0] = gm.astype(jnp.int32)
        minv[0] = jnp.float32(_BIG)
        mini[0] = 0

    buf = lax.rem(i, 2)

    @pl.when(i + 1 < _NBLK)
    def _():
        block_copy(i + 1, lax.rem(i + 1, 2)).start()

    block_copy(i, buf).wait()
    d2 = _block_d2t(bufs.at[buf][...], cent_v[...])

    @pl.when(i < _NBLK - 1)
    def _():
        bmin = jnp.min(d2)

        @pl.when(bmin < minv[0])
        def _():
            minv[0] = bmin
            mini[0] = i

    @pl.when(i == _NBLK - 1)
    def _():
        st_last = jnp.minimum(i * _BLK, _VOCAB - _BLK)
        rr = lax.broadcasted_iota(jnp.int32, (1, _BLK), 1)
        rows = st_last + rr
        bmin = jnp.min(jnp.where(rows < _NSCAN, d2, _BIG))

        @pl.when(bmin < minv[0])
        def _():
            minv[0] = bmin
            mini[0] = i

        wbuf = lax.rem(i + 1, 2)
        wcopy = block_copy(mini[0], wbuf)
        wcopy.start()
        wcopy.wait()
        wd2 = _block_d2t(bufs.at[wbuf][...], cent_v[...])
        wst = jnp.minimum(mini[0] * _BLK, _VOCAB - _BLK)
        wrows = wst + rr
        wd2 = jnp.where(wrows < _NSCAN, wd2, _BIG)
        gmin = minv[0]
        idxo_ref[0, 0] = jnp.min(jnp.where(wd2 == gmin, wrows, _VOCAB))
        dist_ref[0, 0] = jnp.sqrt(gmin)


_main = pl.pallas_call(
    _body,
    grid=(_NBLK,),
    in_specs=[
        pl.BlockSpec(memory_space=pltpu.SMEM),
        pl.BlockSpec(memory_space=pl.ANY),
    ],
    out_specs=(
        pl.BlockSpec(memory_space=pltpu.SMEM),
        pl.BlockSpec(memory_space=pltpu.SMEM),
        pl.BlockSpec(memory_space=pltpu.SMEM),
    ),
    out_shape=(
        jax.ShapeDtypeStruct((1, 1), jnp.int32),
        jax.ShapeDtypeStruct((1, 1), jnp.int32),
        jax.ShapeDtypeStruct((1, 1), jnp.float32),
    ),
    scratch_shapes=[
        pltpu.VMEM((2, _BLK, _DIM), jnp.float32),
        pltpu.VMEM((32, _DIM), jnp.float32),
        pltpu.VMEM((1, _DIM), jnp.float32),
        pltpu.SMEM((1,), jnp.float32),
        pltpu.SMEM((1,), jnp.int32),
        pltpu.SemaphoreType.DMA((2,)),
        pltpu.SemaphoreType.DMA,
    ],
)


def kernel(embeddings, good_idx, bad_idx):
    idx = jnp.concatenate([
        good_idx.astype(jnp.int32),
        bad_idx.astype(jnp.int32),
        jnp.zeros((32 - _TCA,), jnp.int32),
    ])
    clue_len, clue_idx, min_dist = _main(idx, embeddings)
    return clue_idx[0, 0], clue_len[0, 0], min_dist[0, 0]
